# baseline (speedup 1.0000x reference)
import jax
import jax.numpy as jnp
from jax.experimental import pallas as pl
from jax.experimental.pallas import tpu as pltpu

_PH = 4
_PW = 4
_C = 3
_D = 64
_WN = 512 // _PW


def _body(h_ref, w_ref, x0, x1, x2, x3, wt_ref, b_ref, out_ref, idx_ref):
    acc = None
    for pi, xr in enumerate((x0, x1, x2, x3)):
        xb = xr[0, :, 0].reshape(-1, _PW * _C)
        part = jax.lax.dot_general(
            xb, wt_ref[pl.ds(pi * _PW * _C, _PW * _C), :],
            (((1,), (0,)), ((), ())),
            preferred_element_type=jnp.float32)
        acc = part if acc is None else acc + part
    out_ref[0] = acc + b_ref[...]

    n = pl.program_id(0)
    off = (h_ref[n] >> 2) * _WN + (w_ref[n] >> 2)
    p = jax.lax.broadcasted_iota(jnp.int32, (1, 4096), 1)
    idx_ref[0] = (p >> 6) * _WN + (p & 63) + off


def kernel(x, h_offset, w_offset, W, b):
    N, H, Wd, C = x.shape
    h = H // _PH
    w = Wd // _PW
    x4 = x.reshape(N, h, _PH, w, _PW * _C)

    def xspec(pi):
        return pl.BlockSpec((1, h, 1, w, _PW * _C),
                            lambda n, pi=pi: (n, 0, pi, 0, 0))

    emb, idx = pl.pallas_call(
        _body,
        grid=(N,),
        in_specs=[
            pl.BlockSpec(memory_space=pltpu.SMEM),
            pl.BlockSpec(memory_space=pltpu.SMEM),
            xspec(0), xspec(1), xspec(2), xspec(3),
            pl.BlockSpec((_PH * _PW * _C, _D), lambda n: (0, 0)),
            pl.BlockSpec((1, _D), lambda n: (0, 0)),
        ],
        out_specs=[
            pl.BlockSpec((1, h * w, _D), lambda n: (n, 0, 0)),
            pl.BlockSpec((1, 1, h * w), lambda n: (n, 0, 0)),
        ],
        out_shape=[
            jax.ShapeDtypeStruct((N, h * w, _D), jnp.float32),
            jax.ShapeDtypeStruct((N, 1, h * w), jnp.int32),
        ],
    )(h_offset, w_offset, x4, x4, x4, x4, W, b.reshape(1, _D))
    return emb, idx.reshape(N, h * w)


# speedup vs baseline: 1.8773x; 1.8773x over previous
import jax
import jax.numpy as jnp
from jax.experimental import pallas as pl
from jax.experimental.pallas import tpu as pltpu

_D = 64
_WN = 128
_JB = 32
_NJB = 2


def _body(h_ref, w_ref, x_ref, wt_ref, bt_ref, out_ref, idx_ref, bdiag,
          sperm):
    @pl.when(pl.program_id(0) == 0)
    def _build():
        bdiag[...] = jnp.zeros((4 * _NJB, 12 * _JB, _D * _JB), jnp.bfloat16)
        for pi in range(4):
            wblk = wt_ref[pl.ds(pi * 12, 12), :].astype(jnp.bfloat16)
            for hf in range(_NJB):
                for jl in range(_JB):
                    bdiag[pi * _NJB + hf,
                          pl.ds(12 * jl, 12),
                          pl.ds(_D * jl, _D)] = wblk
        rr = jax.lax.broadcasted_iota(jnp.int32, (256, 256), 0)
        cc = jax.lax.broadcasted_iota(jnp.int32, (256, 256), 1)
        sperm[...] = (cc == ((rr & 63) * 4 + (rr >> 6))).astype(jnp.bfloat16)

    xb = x_ref[0].astype(jnp.bfloat16)
    xall = jax.lax.dot_general(
        sperm[...], xb, (((1,), (0,)), ((), ())),
        preferred_element_type=jnp.float32).astype(jnp.bfloat16)
    for hf in range(_NJB):
        acc = None
        for pi in range(4):
            xs = xall[pi * 64:(pi + 1) * 64,
                      hf * 12 * _JB:(hf + 1) * 12 * _JB]
            part = jax.lax.dot_general(
                xs, bdiag[pi * _NJB + hf],
                (((1,), (0,)), ((), ())),
                preferred_element_type=jnp.float32)
            acc = part if acc is None else acc + part
        out_ref[0, :, pl.ds(hf * _D * _JB, _D * _JB)] = (
            acc + bt_ref[:, pl.ds(hf * _D * _JB, _D * _JB)])

    n = pl.program_id(0)
    off = (h_ref[n] >> 2) * _WN + (w_ref[n] >> 2)
    pq = jax.lax.broadcasted_iota(jnp.int32, (1, 4096), 1)
    idx_ref[0] = (pq >> 6) * _WN + (pq & 63) + off


def kernel(x, h_offset, w_offset, W, b):
    N, H, Wd, C = x.shape
    h = H // 4
    w = Wd // 4
    x3 = x.reshape(N, H, Wd * C)
    bt = jnp.tile(b, w).reshape(1, w * _D)

    emb, idx = pl.pallas_call(
        _body,
        grid=(N,),
        in_specs=[
            pl.BlockSpec(memory_space=pltpu.SMEM),
            pl.BlockSpec(memory_space=pltpu.SMEM),
            pl.BlockSpec((1, H, Wd * C), lambda n: (n, 0, 0)),
            pl.BlockSpec((48, _D), lambda n: (0, 0)),
            pl.BlockSpec((1, w * _D), lambda n: (0, 0)),
        ],
        out_specs=[
            pl.BlockSpec((1, h, w * _D), lambda n: (n, 0, 0)),
            pl.BlockSpec((1, 1, h * w), lambda n: (n, 0, 0)),
        ],
        out_shape=[
            jax.ShapeDtypeStruct((N, h, w * _D), jnp.float32),
            jax.ShapeDtypeStruct((N, 1, h * w), jnp.int32),
        ],
        scratch_shapes=[pltpu.VMEM((4 * _NJB, 12 * _JB, _D * _JB),
                                   jnp.bfloat16),
                        pltpu.VMEM((256, 256), jnp.bfloat16)],
    )(h_offset, w_offset, x3, W, bt)
    return emb.reshape(N, h * w, _D), idx.reshape(N, h * w)


# speedup vs baseline: 2.8405x; 1.5131x over previous
import jax
import jax.numpy as jnp
from jax.experimental import pallas as pl
from jax.experimental.pallas import tpu as pltpu

_D = 64
_WN = 128
_IB = 4
_HF = 2


def _body(h_ref, w_ref, x_ref, sp_ref, bc_ref, bt_ref, out_ref, idx_ref):
    pieces = {}
    for img in range(_IB):
        for c in range(3):
            xcb = x_ref[img, c].astype(jnp.bfloat16)
            xac = jax.lax.dot_general(
                sp_ref[...], xcb, (((1,), (0,)), ((), ())),
                preferred_element_type=jnp.float32).astype(jnp.bfloat16)
            for pi in range(4):
                for hf in range(_HF):
                    pieces[(img, c, pi, hf)] = jax.lax.slice(
                        xac, (pi * 64, hf * 128),
                        (pi * 64 + 64, hf * 128 + 128))

    for hf in range(_HF):
        rows = []
        for img in range(_IB):
            rows.append(jnp.concatenate(
                [pieces[(img, c, pi, hf)] for c in range(3)
                 for pi in range(4)], axis=1))
        lhs = jnp.concatenate(rows, axis=0)
        out4 = jax.lax.dot_general(
            lhs, bc_ref[...], (((1,), (0,)), ((), ())),
            preferred_element_type=jnp.float32)
        out4 = out4 + bt_ref[:, pl.ds(hf * 2048, 2048)]
        out_ref[:, :, pl.ds(hf * 2048, 2048)] = out4.reshape(_IB, 64, 2048)

    pq = jax.lax.broadcasted_iota(jnp.int32, (1, 4096), 1)
    base = (pq >> 6) * _WN + (pq & 63)
    for img in range(_IB):
        n = pl.program_id(0) * _IB + img
        off = (h_ref[n] >> 2) * _WN + (w_ref[n] >> 2)
        idx_ref[img] = base + off


def kernel(x, h_offset, w_offset, W, b):
    N, H, Wd, C = x.shape
    h = H // 4
    w = Wd // 4
    xp = x.transpose(0, 3, 1, 2)

    rr = jnp.arange(256, dtype=jnp.int32)[:, None]
    cc = jnp.arange(256, dtype=jnp.int32)[None, :]
    sperm = (cc == (rr & 63) * 4 + (rr >> 6)).astype(jnp.bfloat16)

    w4 = W.reshape(4, 4, 3, _D)
    eye32 = jnp.eye(32, dtype=jnp.float32)
    bcat = jnp.einsum('pqco,jk->cpjqko', w4, eye32)
    bcat = bcat.reshape(1536, 2048).astype(jnp.bfloat16)

    bt = jnp.tile(b, w).reshape(1, w * _D)

    emb2, idx = pl.pallas_call(
        _body,
        grid=(N // _IB,),
        in_specs=[
            pl.BlockSpec(memory_space=pltpu.SMEM),
            pl.BlockSpec(memory_space=pltpu.SMEM),
            pl.BlockSpec((_IB, 3, H, Wd), lambda g: (g, 0, 0, 0)),
            pl.BlockSpec((256, 256), lambda g: (0, 0)),
            pl.BlockSpec((1536, 2048), lambda g: (0, 0)),
            pl.BlockSpec((1, w * _D), lambda g: (0, 0)),
        ],
        out_specs=[
            pl.BlockSpec((_IB, h, w * _D), lambda g: (g, 0, 0)),
            pl.BlockSpec((_IB, 1, h * w), lambda g: (g, 0, 0)),
        ],
        out_shape=[
            jax.ShapeDtypeStruct((N, h, w * _D), jnp.float32),
            jax.ShapeDtypeStruct((N, 1, h * w), jnp.int32),
        ],
    )(h_offset, w_offset, xp, sperm, bcat, bt)
    return emb2.reshape(N, h * w, _D), idx.reshape(N, h * w)
